# baseline (speedup 1.0000x reference)
import jax
import jax.numpy as jnp
from jax import lax
from jax.experimental import pallas as pl
from jax.experimental.pallas import tpu as pltpu
from jax.experimental.pallas import tpu_sc as plsc

B = 16384
L = 20
E = 64
V = 1000000
NC = 2
NS = 16
NW = NC * NS
BPW = B // NW
C = 16
STEPS = BPW // C


def _body(thalf_hbm, tcol_hbm, chalf_hbm, ccol_hbm, temb_hbm, cemb_hbm,
          out_hbm, thalf_v, tcol_v, chalf_v, ccol_v, h_v, u_v, ob_v,
          sem_h, sem_u):
    wid = lax.axis_index("s") * NC + lax.axis_index("c")
    liota = lax.iota(jnp.int32, 16)
    rowbase = [liota * L + l for l in range(L)]

    def step_fn(step, _):
        b0 = wid * BPW + step * C
        pltpu.sync_copy(thalf_hbm.at[pl.ds(b0, C)], thalf_v)
        pltpu.sync_copy(tcol_hbm.at[pl.ds(b0, C)], tcol_v)
        pltpu.sync_copy(chalf_hbm.at[pl.ds(b0 * L, C * L)], chalf_v)
        pltpu.sync_copy(ccol_hbm.at[pl.ds(b0 * L, C * L)], ccol_v)
        cp_h = pltpu.async_copy(temb_hbm.at[thalf_v], h_v, sem_h)
        cp_u = pltpu.async_copy(cemb_hbm.at[chalf_v], u_v, sem_u)
        cp_h.wait()
        cp_u.wait()

        tcol = plsc.load_gather(tcol_v, [liota])
        ccol = [plsc.load_gather(ccol_v, [rowbase[l]]) for l in range(L)]

        def estep(e, accs):
            he = plsc.load_gather(h_v, [liota, tcol + e])
            return tuple(
                acc + he * plsc.load_gather(u_v, [rowbase[l], ccol[l] + e])
                for l, acc in enumerate(accs)
            )

        accs = lax.fori_loop(
            0, E, estep,
            tuple(jnp.zeros((16,), jnp.float32) for _ in range(L)))

        for l in range(L):
            sig = 1.0 / (1.0 + jnp.exp(-accs[l]))
            plsc.store_scatter(ob_v, [rowbase[l]], sig)

        pltpu.sync_copy(ob_v, out_hbm.at[pl.ds(b0 * L, C * L)])
        return ()

    lax.fori_loop(0, STEPS, step_fn, ())


@jax.jit
def _run(thalf, tcol, chalf, ccol, temb2, cemb2):
    mesh = plsc.VectorSubcoreMesh(
        core_axis_name="c", subcore_axis_name="s",
        num_cores=NC, num_subcores=NS)
    f = pl.kernel(
        _body,
        out_type=jax.ShapeDtypeStruct((B * L,), jnp.float32),
        mesh=mesh,
        scratch_types=[
            pltpu.VMEM((C,), jnp.int32),
            pltpu.VMEM((C,), jnp.int32),
            pltpu.VMEM((C * L,), jnp.int32),
            pltpu.VMEM((C * L,), jnp.int32),
            pltpu.VMEM((C, 2 * E), jnp.float32),
            pltpu.VMEM((C * L, 2 * E), jnp.float32),
            pltpu.VMEM((C * L,), jnp.float32),
            pltpu.SemaphoreType.DMA,
            pltpu.SemaphoreType.DMA,
        ],
        compiler_params=pltpu.CompilerParams(
            needs_layout_passes=False, use_tc_tiling_on_sc=False),
    )
    return f(thalf, tcol, chalf, ccol, temb2, cemb2)


def kernel(target_word_id, context_word_ids, target_embeddings,
           context_embeddings):
    tid = target_word_id.reshape(-1).astype(jnp.int32)
    cid = context_word_ids.reshape(-1).astype(jnp.int32)
    thalf = tid >> 1
    tcol = (tid & 1) * E
    chalf = cid >> 1
    ccol = (cid & 1) * E
    temb2 = target_embeddings.reshape(V // 2, 2 * E)
    cemb2 = context_embeddings.reshape(V // 2, 2 * E)
    out = _run(thalf, tcol, chalf, ccol, temb2, cemb2)
    return out.reshape(B, L)


# speedup vs baseline: 1.1800x; 1.1800x over previous
import jax
import jax.numpy as jnp
from jax import lax
from jax.experimental import pallas as pl
from jax.experimental.pallas import tpu as pltpu
from jax.experimental.pallas import tpu_sc as plsc

B = 16384
L = 20
E = 64
V = 1000000
NC = 2
NS = 16
NW = NC * NS
BPW = B // NW
C = 16
STEPS = BPW // C


def _body(thalf_hbm, tcol_hbm, chalf_hbm, ccol_hbm, temb_hbm, cemb_hbm,
          out_hbm, thalf_v, tcol_v, chalf_v, ccol_v, h_v, u_v, ob_v,
          sem_h, sem_u):
    wid = lax.axis_index("s") * NC + lax.axis_index("c")
    liota = lax.iota(jnp.int32, 16)
    rowbase = [liota * L + l for l in range(L)]

    def step_fn(step, _):
        b0 = wid * BPW + step * C
        pltpu.sync_copy(thalf_hbm.at[pl.ds(b0, C)], thalf_v)
        pltpu.sync_copy(tcol_hbm.at[pl.ds(b0, C)], tcol_v)
        pltpu.sync_copy(chalf_hbm.at[pl.ds(b0 * L, C * L)], chalf_v)
        pltpu.sync_copy(ccol_hbm.at[pl.ds(b0 * L, C * L)], ccol_v)
        cp_h = pltpu.async_copy(temb_hbm.at[thalf_v], h_v, sem_h)
        cp_u = pltpu.async_copy(cemb_hbm.at[chalf_v], u_v, sem_u)
        cp_h.wait()
        cp_u.wait()

        tcol = plsc.load_gather(tcol_v, [liota])
        ccol = [plsc.load_gather(ccol_v, [rowbase[l]]) for l in range(L)]

        def estep(e, accs):
            eidx = (liota + e) & (E - 1)
            he = plsc.load_gather(h_v, [liota, tcol + eidx])
            return tuple(
                acc + he * plsc.load_gather(u_v, [rowbase[l], ccol[l] + eidx])
                for l, acc in enumerate(accs)
            )

        accs = lax.fori_loop(
            0, E, estep,
            tuple(jnp.zeros((16,), jnp.float32) for _ in range(L)))

        for l in range(L):
            sig = 1.0 / (1.0 + jnp.exp(-accs[l]))
            plsc.store_scatter(ob_v, [rowbase[l]], sig)

        pltpu.sync_copy(ob_v, out_hbm.at[pl.ds(b0 * L, C * L)])
        return ()

    lax.fori_loop(0, STEPS, step_fn, ())


@jax.jit
def _run(thalf, tcol, chalf, ccol, temb2, cemb2):
    mesh = plsc.VectorSubcoreMesh(
        core_axis_name="c", subcore_axis_name="s",
        num_cores=NC, num_subcores=NS)
    f = pl.kernel(
        _body,
        out_type=jax.ShapeDtypeStruct((B * L,), jnp.float32),
        mesh=mesh,
        scratch_types=[
            pltpu.VMEM((C,), jnp.int32),
            pltpu.VMEM((C,), jnp.int32),
            pltpu.VMEM((C * L,), jnp.int32),
            pltpu.VMEM((C * L,), jnp.int32),
            pltpu.VMEM((C, 2 * E), jnp.float32),
            pltpu.VMEM((C * L, 2 * E), jnp.float32),
            pltpu.VMEM((C * L,), jnp.float32),
            pltpu.SemaphoreType.DMA,
            pltpu.SemaphoreType.DMA,
        ],
        compiler_params=pltpu.CompilerParams(
            needs_layout_passes=False, use_tc_tiling_on_sc=True),
    )
    return f(thalf, tcol, chalf, ccol, temb2, cemb2)


def kernel(target_word_id, context_word_ids, target_embeddings,
           context_embeddings):
    tid = target_word_id.reshape(-1).astype(jnp.int32)
    cid = context_word_ids.reshape(-1).astype(jnp.int32)
    thalf = tid >> 1
    tcol = (tid & 1) * E
    chalf = cid >> 1
    ccol = (cid & 1) * E
    temb2 = target_embeddings.reshape(V // 2, 2 * E)
    cemb2 = context_embeddings.reshape(V // 2, 2 * E)
    out = _run(thalf, tcol, chalf, ccol, temb2, cemb2)
    return out.reshape(B, L)
